# baseline (speedup 1.0000x reference)
import numpy as np
import jax
import jax.numpy as jnp
from jax import lax
from jax.experimental import pallas as pl
from jax.experimental.pallas import tpu as pltpu

B = 128
V = 100000
C = 2048
NC = (V + C - 1) // C

_ROT_A = (13, 15, 26, 6)
_ROT_B = (17, 29, 16, 24)
_KS0 = np.uint32(0)
_KS1 = np.uint32(42)
_KS2 = np.uint32(np.uint32(0x1BD11BDA) ^ np.uint32(42))


def _rotl(x, r):
    r = np.uint32(r)
    return lax.shift_left(x, r) | lax.shift_right_logical(x, np.uint32(32 - r))


def _threefry_bits(p):
    x0 = jnp.zeros_like(p) + _KS0
    x1 = p + _KS1

    def rounds(x0, x1, rots):
        for r in rots:
            x0 = x0 + x1
            x1 = _rotl(x1, r)
            x1 = x0 ^ x1
        return x0, x1

    x0, x1 = rounds(x0, x1, _ROT_A)
    x0, x1 = x0 + _KS1, x1 + (_KS2 + np.uint32(1))
    x0, x1 = rounds(x0, x1, _ROT_B)
    x0, x1 = x0 + _KS2, x1 + (_KS0 + np.uint32(2))
    x0, x1 = rounds(x0, x1, _ROT_A)
    x0, x1 = x0 + _KS0, x1 + (_KS1 + np.uint32(3))
    x0, x1 = rounds(x0, x1, _ROT_B)
    x0, x1 = x0 + _KS1, x1 + (_KS2 + np.uint32(4))
    x0, x1 = rounds(x0, x1, _ROT_A)
    x0, x1 = x0 + _KS2, x1 + (_KS0 + np.uint32(5))
    return x0 ^ x1


def _gumbel(p):
    bits = _threefry_bits(p)
    fb = lax.shift_right_logical(bits, np.uint32(9)) | np.uint32(0x3F800000)
    f = lax.bitcast_convert_type(fb, jnp.float32) - jnp.float32(1.0)
    span = np.float32(np.float32(1.0) - np.float32(1e-20))
    u = jnp.maximum(np.float32(1e-20), f * span + np.float32(1e-20))
    return -jnp.log(-jnp.log(u))


def _sample_kernel(x_ref, o_ref, m_ref, i_ref):
    pid = pl.program_id(0)

    @pl.when(pid == 0)
    def _():
        m_ref[...] = jnp.full((B, 1), -jnp.inf, jnp.float32)
        i_ref[...] = jnp.zeros((B, 1), jnp.int32)

    row = lax.broadcasted_iota(jnp.int32, (B, C), 0)
    col = pid * C + lax.broadcasted_iota(jnp.int32, (B, C), 1)
    p = (row * V + col).astype(jnp.uint32)

    v = x_ref[...] + _gumbel(p)
    v = jnp.where(col < V, v, -jnp.inf)

    cm = jnp.max(v, axis=1, keepdims=True)
    cidx = jnp.min(
        jnp.where(v == cm, col, jnp.int32(np.iinfo(np.int32).max)),
        axis=1, keepdims=True)

    better = cm > m_ref[...]
    i_ref[...] = jnp.where(better, cidx, i_ref[...])
    m_ref[...] = jnp.where(better, cm, m_ref[...])

    @pl.when(pid == NC - 1)
    def _():
        o_ref[...] = i_ref[...]


def kernel(logits):
    out = pl.pallas_call(
        _sample_kernel,
        grid=(NC,),
        in_specs=[pl.BlockSpec((B, C), lambda i: (0, i))],
        out_specs=pl.BlockSpec((B, 1), lambda i: (0, 0)),
        out_shape=jax.ShapeDtypeStruct((B, 1), jnp.int32),
        scratch_shapes=[
            pltpu.VMEM((B, 1), jnp.float32),
            pltpu.VMEM((B, 1), jnp.int32),
        ],
    )(logits)
    return out[:, 0].astype(jnp.int64)
